# baseline (speedup 1.0000x reference)
import functools

import jax
import jax.numpy as jnp
from jax import lax
from jax.experimental import pallas as pl
from jax.experimental.pallas import tpu as pltpu
from jax.experimental.pallas import tpu_sc as plsc

_B = 4
_S = 8192
_D = 1024
_NC = 2
_NS = 16
_NW = _NC * _NS
_ROWS_PER_W = _S // _NW
_CH = 64

_mesh = plsc.VectorSubcoreMesh(core_axis_name="c", subcore_axis_name="s")


@functools.partial(
    pl.kernel,
    out_type=jax.ShapeDtypeStruct((1, _S, _D), jnp.float32),
    mesh=_mesh,
    scratch_types=[pltpu.VMEM((_CH, _D), jnp.float32)],
)
def _sc_part(pe_hbm, out_hbm, buf):
    wid = lax.axis_index("s") * _NC + lax.axis_index("c")
    base = wid * _ROWS_PER_W

    def chunk(i, carry):
        row0 = base + i * _CH
        pltpu.sync_copy(pe_hbm.at[pl.ds(row0, _CH)], buf)
        pltpu.sync_copy(buf, out_hbm.at[0, pl.ds(row0, _CH)])
        return carry

    lax.fori_loop(0, _ROWS_PER_W // _CH, chunk, 0)


_BLK = 256


def _tc_body(pe_ref, out_ref):
    out_ref[...] = jnp.broadcast_to(pe_ref[...][None], (_B - 1, _BLK, _D))


_tc_part = pl.pallas_call(
    _tc_body,
    grid=(_S // _BLK,),
    in_specs=[pl.BlockSpec((_BLK, _D), lambda i: (i, 0))],
    out_specs=pl.BlockSpec((_B - 1, _BLK, _D), lambda i: (0, i, 0)),
    out_shape=jax.ShapeDtypeStruct((_B - 1, _S, _D), jnp.float32),
)


def kernel(x, pe):
    del x
    a = _sc_part(pe)
    b = _tc_part(pe)
    return jnp.concatenate([a, b], axis=0)


# speedup vs baseline: 2.2298x; 2.2298x over previous
import functools

import jax
import jax.numpy as jnp
from jax import lax
from jax.experimental import pallas as pl
from jax.experimental.pallas import tpu as pltpu
from jax.experimental.pallas import tpu_sc as plsc

_B = 4
_S = 8192
_D = 1024
_NC = 2
_NS = 16
_NW = _NC * _NS
_ROWS_PER_W = _S // _NW
_CH = 32
_N = _ROWS_PER_W // _CH

_mesh = plsc.VectorSubcoreMesh(core_axis_name="c", subcore_axis_name="s")


@functools.partial(
    pl.kernel,
    out_type=jax.ShapeDtypeStruct((_B, _S, _D), jnp.float32),
    mesh=_mesh,
    scratch_types=[
        pltpu.VMEM((2, _CH, _D), jnp.float32),
        pltpu.SemaphoreType.DMA,
        pltpu.SemaphoreType.DMA,
        pltpu.SemaphoreType.DMA,
        pltpu.SemaphoreType.DMA,
    ],
)
def _pe_broadcast(pe_hbm, out_hbm, buf, si0, si1, so0, so1):
    wid = lax.axis_index("s") * _NC + lax.axis_index("c")
    base = wid * _ROWS_PER_W
    sin = (si0, si1)
    sout = (so0, so1)

    def start_in(i):
        slot = i % 2
        return pltpu.async_copy(
            pe_hbm.at[pl.ds(base + i * _CH, _CH)], buf.at[slot], sin[slot])

    def start_outs(i):
        slot = i % 2
        return [
            pltpu.async_copy(
                buf.at[slot], out_hbm.at[b, pl.ds(base + i * _CH, _CH)],
                sout[slot])
            for b in range(_B)
        ]

    cin = [None] * _N
    couts = [None] * _N
    cin[0] = start_in(0)
    for i in range(_N):
        if i + 1 < _N:
            if i >= 1:
                for c in couts[i - 1]:
                    c.wait()
            cin[i + 1] = start_in(i + 1)
        cin[i].wait()
        couts[i] = start_outs(i)
    for c in couts[_N - 2] + couts[_N - 1]:
        c.wait()


def kernel(x, pe):
    del x
    return _pe_broadcast(pe)
